# baseline (speedup 1.0000x reference)
import functools

import jax
import jax.numpy as jnp
from jax import lax
from jax.experimental import pallas as pl
from jax.experimental.pallas import tpu as pltpu

NEG_SLOPE = 0.2
OUT_FEATS = 16
NB = 2


def _seg_masks(rows, seg):
    t = lax.broadcasted_iota(jnp.int32, (rows, 1), 0) % seg
    return (t != 0).astype(jnp.float32), (t != seg - 1).astype(jnp.float32)


def _cnr3(h_bf, wcat, scale, bias, m_nf, m_nl):
    rows, C = h_bf.shape
    y = jnp.dot(h_bf, wcat, preferred_element_type=jnp.float32)
    z = (y[:, C:2 * C]
         + pltpu.roll(y[:, :C], 1, 0) * m_nf
         + pltpu.roll(y[:, 2 * C:], rows - 1, 0) * m_nl)
    z = z * scale + bias
    return jnp.maximum(z, NEG_SLOPE * z)


def _cnr4_down(h_bf, wcat4, scale, bias, batch, seg):
    rows, C = h_bf.shape
    to = seg // 2
    y = jnp.dot(h_bf, wcat4, preferred_element_type=jnp.float32)
    t = lax.broadcasted_iota(jnp.int32, (rows, 1), 0) % seg
    m0 = (t != 0).astype(jnp.float32)
    m2 = (t != seg - 1).astype(jnp.float32)
    m3 = (t < seg - 2).astype(jnp.float32)
    z = (y[:, C:2 * C]
         + pltpu.roll(y[:, :C], 1, 0) * m0
         + pltpu.roll(y[:, 2 * C:3 * C], rows - 1, 0) * m2
         + pltpu.roll(y[:, 3 * C:], rows - 2, 0) * m3)
    z = z * scale + bias
    z = jnp.maximum(z, NEG_SLOPE * z).astype(jnp.bfloat16)
    r = lax.broadcasted_iota(jnp.int32, (batch * to, rows), 0)
    c = lax.broadcasted_iota(jnp.int32, (batch * to, rows), 1)
    sel = (c == (r // to) * seg + 2 * (r % to)).astype(jnp.bfloat16)
    return jnp.dot(sel, z, preferred_element_type=jnp.float32)


def _up2(h_bf, batch, seg):
    ro, ri = batch * 2 * seg, batch * seg
    r = lax.broadcasted_iota(jnp.int32, (ro, ri), 0)
    c = lax.broadcasted_iota(jnp.int32, (ro, ri), 1)
    up = (c == (r // (2 * seg)) * seg + (r % (2 * seg)) // 2).astype(jnp.bfloat16)
    return jnp.dot(up, h_bf, preferred_element_type=jnp.float32)


def _fused_kernel(xt_ref, lab_ref,
                  tw_ref, ts_ref, tb_ref,
                  pw_ref, ps_ref, pb_ref,
                  dw_ref, ds_ref, db_ref,
                  uw_ref, us_ref, ub_ref,
                  cw_ref, cs_ref, cb_ref, chw_ref, chb_ref,
                  gw_ref, gs_ref, gb_ref, ghw_ref, ghb_ref,
                  so_ref, oo_ref, *, rep, nc):
    Bh, Ts, C = xt_ref.shape
    T = Ts * rep
    rt = Bh * Ts
    rows = Bh * T

    h0 = xt_ref[...].reshape(rt, C).astype(jnp.bfloat16)
    mf_t, ml_t = _seg_masks(rt, Ts)
    r = lax.broadcasted_iota(jnp.int32, (rows, rt), 0)
    c = lax.broadcasted_iota(jnp.int32, (rows, rt), 1)
    up = (c == (r // T) * Ts + (r % T) // rep).astype(jnp.bfloat16)

    G, Lt = tw_ref.shape[0], tw_ref.shape[1]
    x = jnp.zeros((rows, C), jnp.float32)
    for g in range(G):
        h = h0
        for l in range(Lt):
            h = _cnr3(h, tw_ref[g, l], ts_ref[g, l], tb_ref[g, l],
                      mf_t, ml_t).astype(jnp.bfloat16)
        x = x + jnp.dot(up, h, preferred_element_type=jnp.float32) * lab_ref[g]

    m_nf, m_nl = _seg_masks(rows, T)
    h_bf = x.astype(jnp.bfloat16)
    z = None
    for l in range(pw_ref.shape[0]):
        z = _cnr3(h_bf, pw_ref[l], ps_ref[l], pb_ref[l], m_nf, m_nl)
        h_bf = z.astype(jnp.bfloat16)

    n_lvl = dw_ref.shape[0]
    residuals = [z]
    seg = T
    for l in range(n_lvl):
        z = _cnr4_down(h_bf, dw_ref[l], ds_ref[l], db_ref[l], Bh, seg)
        seg //= 2
        h_bf = z.astype(jnp.bfloat16)
        if l < n_lvl - 1:
            residuals.append(z)

    for l in range(n_lvl):
        hu = _up2(h_bf, Bh, seg)
        seg *= 2
        hsum = hu + residuals[n_lvl - 1 - l]
        mf, ml = _seg_masks(Bh * seg, seg)
        z = _cnr3(hsum.astype(jnp.bfloat16), uw_ref[l], us_ref[l], ub_ref[l],
                  mf, ml)
        h_bf = z.astype(jnp.bfloat16)

    hc = h_bf
    for l in range(cw_ref.shape[0]):
        hc = _cnr3(hc, cw_ref[l], cs_ref[l], cb_ref[l], m_nf, m_nl
                   ).astype(jnp.bfloat16)
    score = jnp.dot(hc, chw_ref[...], preferred_element_type=jnp.float32)
    score = score + chb_ref[...]
    Fp = score.shape[-1]
    so_ref[...] = score.reshape(Bh, T, Fp)

    col = lax.broadcasted_iota(jnp.int32, score.shape, 1)
    sm = jnp.where(col < nc, score, -1e30)
    m = jnp.max(sm, axis=-1, keepdims=True)
    e = jnp.where(col < nc, jnp.exp(sm - m), 0.0)
    p = e / jnp.sum(e, axis=-1, keepdims=True)

    Gd, Ld = gw_ref.shape[0], gw_ref.shape[1]
    colp = lax.broadcasted_iota(jnp.int32, p.shape, 1)
    acc = jnp.zeros((rows, Fp), jnp.float32)
    for g in range(Gd):
        h = h_bf
        for l in range(Ld):
            h = _cnr3(h, gw_ref[g, l], gs_ref[g, l], gb_ref[g, l],
                      m_nf, m_nl).astype(jnp.bfloat16)
        y = jnp.dot(h, ghw_ref[g], preferred_element_type=jnp.float32) + ghb_ref[g]
        pg = jnp.sum(jnp.where(colp == g, p, 0.0), axis=-1, keepdims=True)
        acc = acc + y * pg
    oo_ref[...] = acc.reshape(Bh, T, Fp)


def _const_spec(shape):
    rank = len(shape)
    return pl.BlockSpec(tuple(shape), lambda i, _r=rank: (0,) * _r)


def kernel(text_W, text_S, text_B,
           unet_pw, unet_ps, unet_pb, unet_dw, unet_ds, unet_db,
           unet_uw, unet_us, unet_ub,
           cls_w, cls_s, cls_b, cls_hw, cls_hb,
           dec_w, dec_s, dec_b, dec_hw, dec_hb,
           text, labels, labels_pos):
    Bb, Ts, text_ch = text.shape
    T = labels.shape[1]
    rep = T // Ts
    ncp = text_W.shape[0]
    C = text_W.shape[2]
    nc = dec_w.shape[0]
    Fp = dec_hw.shape[-1]
    Bh = Bb // NB

    x_text = jnp.pad(text, ((0, 0), (0, 0), (0, C - text_ch)))
    lab = jax.nn.one_hot(labels_pos, ncp, dtype=jnp.float32)
    lab = jnp.transpose(lab, (2, 0, 1)).reshape(ncp, Bb * T, 1)
    lab = jnp.broadcast_to(lab, (ncp, Bb * T, C)).astype(jnp.bfloat16)

    weights = (text_W, text_S, text_B,
               unet_pw, unet_ps, unet_pb, unet_dw, unet_ds, unet_db,
               unet_uw, unet_us, unet_ub,
               cls_w, cls_s, cls_b, cls_hw, cls_hb,
               dec_w, dec_s, dec_b, dec_hw, dec_hb)

    in_specs = [
        pl.BlockSpec((Bh, Ts, C), lambda i: (i, 0, 0)),
        pl.BlockSpec((ncp, Bh * T, C), lambda i: (0, i, 0)),
    ] + [_const_spec(w.shape) for w in weights]

    score_pad, out_pad = pl.pallas_call(
        functools.partial(_fused_kernel, rep=rep, nc=nc),
        grid=(NB,),
        in_specs=in_specs,
        out_specs=(pl.BlockSpec((Bh, T, Fp), lambda i: (i, 0, 0)),
                   pl.BlockSpec((Bh, T, Fp), lambda i: (i, 0, 0))),
        out_shape=(jax.ShapeDtypeStruct((Bb, T, Fp), jnp.float32),
                   jax.ShapeDtypeStruct((Bb, T, Fp), jnp.float32)),
        compiler_params=pltpu.CompilerParams(
            dimension_semantics=("arbitrary",),
            vmem_limit_bytes=64 * 1024 * 1024,
        ),
    )(x_text, lab, *weights)

    score = score_pad[:, :, :nc]
    logp = jax.nn.log_softmax(score, axis=-1)
    labels_oh = jax.nn.one_hot(labels, nc, dtype=jnp.float32)
    ce_loss = -jnp.mean(jnp.sum(labels_oh * logp, axis=-1))
    return out_pad[:, :, :OUT_FEATS], [ce_loss]


# speedup vs baseline: 1.0508x; 1.0508x over previous
import functools

import jax
import jax.numpy as jnp
from jax import lax
from jax.experimental import pallas as pl
from jax.experimental.pallas import tpu as pltpu

NEG_SLOPE = 0.2
OUT_FEATS = 16
NB = 2


def _seg_masks(rows, seg):
    t = lax.broadcasted_iota(jnp.int32, (rows, 1), 0) % seg
    return (t != 0).astype(jnp.float32), (t != seg - 1).astype(jnp.float32)


def _cnr3(h_bf, wcat, scale, bias, m_nf, m_nl):
    rows, C = h_bf.shape
    y = jnp.dot(h_bf, wcat, preferred_element_type=jnp.float32)
    z = (y[:, C:2 * C]
         + pltpu.roll(y[:, :C], 1, 0) * m_nf
         + pltpu.roll(y[:, 2 * C:], rows - 1, 0) * m_nl)
    z = z * scale + bias
    return jnp.maximum(z, NEG_SLOPE * z)


def _down4s(h_bf, w_pair, scale, bias, scr, seg):
    rows, C = h_bf.shape
    half, to = rows // 2, seg // 2
    h32 = h_bf.astype(jnp.float32)
    nk = C // 128
    for k in range(nk):
        scr[k, pl.ds(0, rows), :] = h32[:, 128 * k:128 * (k + 1)]
    he = jnp.concatenate([scr[k, pl.ds(0, half, 2), :] for k in range(nk)],
                         axis=-1).astype(jnp.bfloat16)
    ho = jnp.concatenate([scr[k, pl.ds(1, half, 2), :] for k in range(nk)],
                         axis=-1).astype(jnp.bfloat16)
    ye = jnp.dot(he, w_pair[:, :2 * C], preferred_element_type=jnp.float32)
    yo = jnp.dot(ho, w_pair[:, 2 * C:], preferred_element_type=jnp.float32)
    mf, ml = _seg_masks(half, to)
    z = (ye[:, :C]
         + pltpu.roll(yo[:, :C], 1, 0) * mf
         + yo[:, C:]
         + pltpu.roll(ye[:, C:], half - 1, 0) * ml)
    z = z * scale + bias
    return jnp.maximum(z, NEG_SLOPE * z).astype(jnp.bfloat16)


def _up2s(h_bf, scr):
    ri, C = h_bf.shape
    h32 = h_bf.astype(jnp.float32)
    nk = C // 128
    for k in range(nk):
        scr[k, pl.ds(0, ri, 2), :] = h32[:, 128 * k:128 * (k + 1)]
        scr[k, pl.ds(1, ri, 2), :] = h32[:, 128 * k:128 * (k + 1)]
    return jnp.concatenate([scr[k, pl.ds(0, 2 * ri), :] for k in range(nk)],
                           axis=-1)


def _fused_kernel(xt_ref, lab_ref,
                  tw_ref, ts_ref, tb_ref,
                  pw_ref, ps_ref, pb_ref,
                  dw_ref, ds_ref, db_ref,
                  uw_ref, us_ref, ub_ref,
                  cw_ref, cs_ref, cb_ref, chw_ref, chb_ref,
                  gw_ref, gs_ref, gb_ref, ghw_ref, ghb_ref,
                  so_ref, oo_ref, scr_ref, *, rep, nc):
    Bh, Ts, C = xt_ref.shape
    T = Ts * rep
    rt = Bh * Ts
    rows = Bh * T

    h0 = xt_ref[...].reshape(rt, C).astype(jnp.bfloat16)
    mf_t, ml_t = _seg_masks(rt, Ts)

    G, Lt = tw_ref.shape[0], tw_ref.shape[1]
    x = jnp.zeros((rows, C), jnp.float32)
    for g in range(G):
        h = h0
        for l in range(Lt):
            h = _cnr3(h, tw_ref[g, l], ts_ref[g, l], tb_ref[g, l],
                      mf_t, ml_t).astype(jnp.bfloat16)
        h32 = h.astype(jnp.float32)
        for k in range(C // 128):
            for j in range(rep):
                scr_ref[k, pl.ds(j, rt, rep), :] = h32[:, 128 * k:128 * (k + 1)]
        y_up = jnp.concatenate(
            [scr_ref[k, pl.ds(0, rows), :] for k in range(C // 128)], axis=-1)
        x = x + y_up * lab_ref[g]

    m_nf, m_nl = _seg_masks(rows, T)
    h_bf = x.astype(jnp.bfloat16)
    z = None
    for l in range(pw_ref.shape[0]):
        z = _cnr3(h_bf, pw_ref[l], ps_ref[l], pb_ref[l], m_nf, m_nl)
        h_bf = z.astype(jnp.bfloat16)

    n_lvl = dw_ref.shape[0]
    residuals = [z]
    seg = T
    for l in range(n_lvl):
        zb = _down4s(h_bf, dw_ref[l], ds_ref[l], db_ref[l], scr_ref, seg)
        seg //= 2
        h_bf = zb
        if l < n_lvl - 1:
            residuals.append(zb.astype(jnp.float32))

    for l in range(n_lvl):
        hu = _up2s(h_bf, scr_ref)
        seg *= 2
        hsum = hu + residuals[n_lvl - 1 - l]
        mf, ml = _seg_masks(Bh * seg, seg)
        z = _cnr3(hsum.astype(jnp.bfloat16), uw_ref[l], us_ref[l], ub_ref[l],
                  mf, ml)
        h_bf = z.astype(jnp.bfloat16)

    hc = h_bf
    for l in range(cw_ref.shape[0]):
        hc = _cnr3(hc, cw_ref[l], cs_ref[l], cb_ref[l], m_nf, m_nl
                   ).astype(jnp.bfloat16)
    score = jnp.dot(hc, chw_ref[...], preferred_element_type=jnp.float32)
    score = score + chb_ref[...]
    Fp = score.shape[-1]
    so_ref[...] = score.reshape(Bh, T, Fp)

    col = lax.broadcasted_iota(jnp.int32, score.shape, 1)
    sm = jnp.where(col < nc, score, -1e30)
    m = jnp.max(sm, axis=-1, keepdims=True)
    e = jnp.where(col < nc, jnp.exp(sm - m), 0.0)
    p = e / jnp.sum(e, axis=-1, keepdims=True)

    Gd, Ld = gw_ref.shape[0], gw_ref.shape[1]
    colp = lax.broadcasted_iota(jnp.int32, p.shape, 1)
    acc = jnp.zeros((rows, Fp), jnp.float32)
    for g in range(Gd):
        h = h_bf
        for l in range(Ld):
            h = _cnr3(h, gw_ref[g, l], gs_ref[g, l], gb_ref[g, l],
                      m_nf, m_nl).astype(jnp.bfloat16)
        y = jnp.dot(h, ghw_ref[g], preferred_element_type=jnp.float32) + ghb_ref[g]
        pg = jnp.sum(jnp.where(colp == g, p, 0.0), axis=-1, keepdims=True)
        acc = acc + y * pg
    oo_ref[...] = acc.reshape(Bh, T, Fp)


def _const_spec(shape):
    rank = len(shape)
    return pl.BlockSpec(tuple(shape), lambda i, _r=rank: (0,) * _r)


def kernel(text_W, text_S, text_B,
           unet_pw, unet_ps, unet_pb, unet_dw, unet_ds, unet_db,
           unet_uw, unet_us, unet_ub,
           cls_w, cls_s, cls_b, cls_hw, cls_hb,
           dec_w, dec_s, dec_b, dec_hw, dec_hb,
           text, labels, labels_pos):
    Bb, Ts, text_ch = text.shape
    T = labels.shape[1]
    rep = T // Ts
    ncp = text_W.shape[0]
    C = text_W.shape[2]
    nc = dec_w.shape[0]
    Fp = dec_hw.shape[-1]
    Bh = Bb // NB

    x_text = jnp.pad(text, ((0, 0), (0, 0), (0, C - text_ch)))
    lab = jax.nn.one_hot(labels_pos, ncp, dtype=jnp.float32)
    lab = jnp.transpose(lab, (2, 0, 1)).reshape(ncp, Bb * T, 1)
    lab = jnp.broadcast_to(lab, (ncp, Bb * T, C)).astype(jnp.bfloat16)
    unet_dw = jnp.concatenate(
        [unet_dw[:, :, C:2 * C], unet_dw[:, :, 3 * C:],
         unet_dw[:, :, :C], unet_dw[:, :, 2 * C:3 * C]], axis=-1)

    weights = (text_W, text_S, text_B,
               unet_pw, unet_ps, unet_pb, unet_dw, unet_ds, unet_db,
               unet_uw, unet_us, unet_ub,
               cls_w, cls_s, cls_b, cls_hw, cls_hb,
               dec_w, dec_s, dec_b, dec_hw, dec_hb)

    in_specs = [
        pl.BlockSpec((Bh, Ts, C), lambda i: (i, 0, 0)),
        pl.BlockSpec((ncp, Bh * T, C), lambda i: (0, i, 0)),
    ] + [_const_spec(w.shape) for w in weights]

    score_pad, out_pad = pl.pallas_call(
        functools.partial(_fused_kernel, rep=rep, nc=nc),
        grid=(NB,),
        in_specs=in_specs,
        out_specs=(pl.BlockSpec((Bh, T, Fp), lambda i: (i, 0, 0)),
                   pl.BlockSpec((Bh, T, Fp), lambda i: (i, 0, 0))),
        out_shape=(jax.ShapeDtypeStruct((Bb, T, Fp), jnp.float32),
                   jax.ShapeDtypeStruct((Bb, T, Fp), jnp.float32)),
        scratch_shapes=[pltpu.VMEM((C // 128, Bh * T, 128), jnp.float32)],
        compiler_params=pltpu.CompilerParams(
            dimension_semantics=("arbitrary",),
            vmem_limit_bytes=64 * 1024 * 1024,
        ),
    )(x_text, lab, *weights)

    score = score_pad[:, :, :nc]
    logp = jax.nn.log_softmax(score, axis=-1)
    labels_oh = jax.nn.one_hot(labels, nc, dtype=jnp.float32)
    ce_loss = -jnp.mean(jnp.sum(labels_oh * logp, axis=-1))
    return out_pad[:, :, :OUT_FEATS], [ce_loss]


# speedup vs baseline: 1.0893x; 1.0366x over previous
import functools

import jax
import jax.numpy as jnp
from jax import lax
from jax.experimental import pallas as pl
from jax.experimental.pallas import tpu as pltpu

NEG_SLOPE = 0.2
OUT_FEATS = 16
NB = 1


def _seg_masks(rows, seg):
    t = lax.broadcasted_iota(jnp.int32, (rows, 1), 0) % seg
    return (t != 0).astype(jnp.float32), (t != seg - 1).astype(jnp.float32)


def _cnr3(h_bf, wcat, scale, bias, m_nf, m_nl):
    rows, C = h_bf.shape
    y = jnp.dot(h_bf, wcat, preferred_element_type=jnp.float32)
    z = (y[:, C:2 * C]
         + pltpu.roll(y[:, :C], 1, 0) * m_nf
         + pltpu.roll(y[:, 2 * C:], rows - 1, 0) * m_nl)
    z = z * scale + bias
    return jnp.maximum(z, NEG_SLOPE * z)


def _down4s(h_bf, w_pair, scale, bias, scr, seg):
    rows, C = h_bf.shape
    half, to = rows // 2, seg // 2
    h32 = h_bf.astype(jnp.float32)
    nk = C // 128
    for k in range(nk):
        scr[k, pl.ds(0, rows), :] = h32[:, 128 * k:128 * (k + 1)]
    he = jnp.concatenate([scr[k, pl.ds(0, half, 2), :] for k in range(nk)],
                         axis=-1).astype(jnp.bfloat16)
    ho = jnp.concatenate([scr[k, pl.ds(1, half, 2), :] for k in range(nk)],
                         axis=-1).astype(jnp.bfloat16)
    ye = jnp.dot(he, w_pair[:, :2 * C], preferred_element_type=jnp.float32)
    yo = jnp.dot(ho, w_pair[:, 2 * C:], preferred_element_type=jnp.float32)
    mf, ml = _seg_masks(half, to)
    z = (ye[:, :C]
         + pltpu.roll(yo[:, :C], 1, 0) * mf
         + yo[:, C:]
         + pltpu.roll(ye[:, C:], half - 1, 0) * ml)
    z = z * scale + bias
    return jnp.maximum(z, NEG_SLOPE * z).astype(jnp.bfloat16)


def _up2s(h_bf, scr):
    ri, C = h_bf.shape
    h32 = h_bf.astype(jnp.float32)
    nk = C // 128
    for k in range(nk):
        scr[k, pl.ds(0, ri, 2), :] = h32[:, 128 * k:128 * (k + 1)]
        scr[k, pl.ds(1, ri, 2), :] = h32[:, 128 * k:128 * (k + 1)]
    return jnp.concatenate([scr[k, pl.ds(0, 2 * ri), :] for k in range(nk)],
                           axis=-1)


def _fused_kernel(xt_ref, lab_ref,
                  tw_ref, ts_ref, tb_ref,
                  pw_ref, ps_ref, pb_ref,
                  dw_ref, ds_ref, db_ref,
                  uw_ref, us_ref, ub_ref,
                  cw_ref, cs_ref, cb_ref, chw_ref, chb_ref,
                  gw_ref, gs_ref, gb_ref, ghw_ref, ghb_ref,
                  so_ref, oo_ref, scr_ref, *, rep, nc):
    Bh, Ts, C = xt_ref.shape
    T = Ts * rep
    rt = Bh * Ts
    rows = Bh * T

    h0 = xt_ref[...].reshape(rt, C).astype(jnp.bfloat16)
    mf_t, ml_t = _seg_masks(rt, Ts)

    G, Lt = tw_ref.shape[0], tw_ref.shape[1]
    x = jnp.zeros((rows, C), jnp.float32)
    for g in range(G):
        h = h0
        for l in range(Lt):
            h = _cnr3(h, tw_ref[g, l], ts_ref[g, l], tb_ref[g, l],
                      mf_t, ml_t).astype(jnp.bfloat16)
        h32 = h.astype(jnp.float32)
        for k in range(C // 128):
            for j in range(rep):
                scr_ref[k, pl.ds(j, rt, rep), :] = h32[:, 128 * k:128 * (k + 1)]
        y_up = jnp.concatenate(
            [scr_ref[k, pl.ds(0, rows), :] for k in range(C // 128)], axis=-1)
        x = x + y_up * lab_ref[g]

    m_nf, m_nl = _seg_masks(rows, T)
    h_bf = x.astype(jnp.bfloat16)
    z = None
    for l in range(pw_ref.shape[0]):
        z = _cnr3(h_bf, pw_ref[l], ps_ref[l], pb_ref[l], m_nf, m_nl)
        h_bf = z.astype(jnp.bfloat16)

    n_lvl = dw_ref.shape[0]
    residuals = [z]
    seg = T
    for l in range(n_lvl):
        zb = _down4s(h_bf, dw_ref[l], ds_ref[l], db_ref[l], scr_ref, seg)
        seg //= 2
        h_bf = zb
        if l < n_lvl - 1:
            residuals.append(zb.astype(jnp.float32))

    for l in range(n_lvl):
        hu = _up2s(h_bf, scr_ref)
        seg *= 2
        hsum = hu + residuals[n_lvl - 1 - l]
        mf, ml = _seg_masks(Bh * seg, seg)
        z = _cnr3(hsum.astype(jnp.bfloat16), uw_ref[l], us_ref[l], ub_ref[l],
                  mf, ml)
        h_bf = z.astype(jnp.bfloat16)

    hc = h_bf
    for l in range(cw_ref.shape[0]):
        hc = _cnr3(hc, cw_ref[l], cs_ref[l], cb_ref[l], m_nf, m_nl
                   ).astype(jnp.bfloat16)
    score = jnp.dot(hc, chw_ref[...], preferred_element_type=jnp.float32)
    score = score + chb_ref[...]
    Fp = score.shape[-1]
    so_ref[...] = score.reshape(Bh, T, Fp)

    col = lax.broadcasted_iota(jnp.int32, score.shape, 1)
    sm = jnp.where(col < nc, score, -1e30)
    m = jnp.max(sm, axis=-1, keepdims=True)
    e = jnp.where(col < nc, jnp.exp(sm - m), 0.0)
    p = e / jnp.sum(e, axis=-1, keepdims=True)

    Gd, Ld = gw_ref.shape[0], gw_ref.shape[1]
    colp = lax.broadcasted_iota(jnp.int32, p.shape, 1)
    acc = jnp.zeros((rows, Fp), jnp.float32)
    for g in range(Gd):
        h = h_bf
        for l in range(Ld):
            h = _cnr3(h, gw_ref[g, l], gs_ref[g, l], gb_ref[g, l],
                      m_nf, m_nl).astype(jnp.bfloat16)
        y = jnp.dot(h, ghw_ref[g], preferred_element_type=jnp.float32) + ghb_ref[g]
        pg = jnp.sum(jnp.where(colp == g, p, 0.0), axis=-1, keepdims=True)
        acc = acc + y * pg
    oo_ref[...] = acc.reshape(Bh, T, Fp)


def _const_spec(shape):
    rank = len(shape)
    return pl.BlockSpec(tuple(shape), lambda i, _r=rank: (0,) * _r)


def kernel(text_W, text_S, text_B,
           unet_pw, unet_ps, unet_pb, unet_dw, unet_ds, unet_db,
           unet_uw, unet_us, unet_ub,
           cls_w, cls_s, cls_b, cls_hw, cls_hb,
           dec_w, dec_s, dec_b, dec_hw, dec_hb,
           text, labels, labels_pos):
    Bb, Ts, text_ch = text.shape
    T = labels.shape[1]
    rep = T // Ts
    ncp = text_W.shape[0]
    C = text_W.shape[2]
    nc = dec_w.shape[0]
    Fp = dec_hw.shape[-1]
    Bh = Bb // NB

    x_text = jnp.pad(text, ((0, 0), (0, 0), (0, C - text_ch)))
    lab = jax.nn.one_hot(labels_pos, ncp, dtype=jnp.float32)
    lab = jnp.transpose(lab, (2, 0, 1)).reshape(ncp, Bb * T, 1)
    lab = jnp.broadcast_to(lab, (ncp, Bb * T, C)).astype(jnp.bfloat16)
    unet_dw = jnp.concatenate(
        [unet_dw[:, :, C:2 * C], unet_dw[:, :, 3 * C:],
         unet_dw[:, :, :C], unet_dw[:, :, 2 * C:3 * C]], axis=-1)

    weights = (text_W, text_S, text_B,
               unet_pw, unet_ps, unet_pb, unet_dw, unet_ds, unet_db,
               unet_uw, unet_us, unet_ub,
               cls_w, cls_s, cls_b, cls_hw, cls_hb,
               dec_w, dec_s, dec_b, dec_hw, dec_hb)

    in_specs = [
        pl.BlockSpec((Bh, Ts, C), lambda i: (i, 0, 0)),
        pl.BlockSpec((ncp, Bh * T, C), lambda i: (0, i, 0)),
    ] + [_const_spec(w.shape) for w in weights]

    score_pad, out_pad = pl.pallas_call(
        functools.partial(_fused_kernel, rep=rep, nc=nc),
        grid=(NB,),
        in_specs=in_specs,
        out_specs=(pl.BlockSpec((Bh, T, Fp), lambda i: (i, 0, 0)),
                   pl.BlockSpec((Bh, T, Fp), lambda i: (i, 0, 0))),
        out_shape=(jax.ShapeDtypeStruct((Bb, T, Fp), jnp.float32),
                   jax.ShapeDtypeStruct((Bb, T, Fp), jnp.float32)),
        scratch_shapes=[pltpu.VMEM((C // 128, Bh * T, 128), jnp.float32)],
        compiler_params=pltpu.CompilerParams(
            dimension_semantics=("arbitrary",),
            vmem_limit_bytes=64 * 1024 * 1024,
        ),
    )(x_text, lab, *weights)

    score = score_pad[:, :, :nc]
    logp = jax.nn.log_softmax(score, axis=-1)
    labels_oh = jax.nn.one_hot(labels, nc, dtype=jnp.float32)
    ce_loss = -jnp.mean(jnp.sum(labels_oh * logp, axis=-1))
    return out_pad[:, :, :OUT_FEATS], [ce_loss]
